# Initial kernel scaffold; baseline (speedup 1.0000x reference)
import jax
import jax.numpy as jnp
from jax.experimental import pallas as pl


def kernel(orders, freqs_cis, W1, b1, W2, b2):
    raise NotImplementedError("write your pallas kernel here")


# speedup vs baseline: 4.6913x; 4.6913x over previous
import functools

import jax
import jax.numpy as jnp
from jax import lax
from jax.experimental import pallas as pl
from jax.experimental.pallas import tpu as pltpu
from jax.experimental.pallas import tpu_sc as plsc

D_IN = 768
D_OUT = 1024


_BLK = 512


def _mlp_body(x_ref, w1_ref, b1_ref, w2_ref, b2_ref, o_ref):
    h = jnp.dot(x_ref[...], w1_ref[...], preferred_element_type=jnp.float32)
    h = h + b1_ref[...]
    h = h * jax.nn.sigmoid(h)
    o = jnp.dot(h, w2_ref[...], preferred_element_type=jnp.float32)
    o_ref[...] = o + b2_ref[...]


def _mlp_table(freqs, W1, b1, W2, b2):
    n = freqs.shape[0]
    grid = pl.cdiv(n, _BLK)
    return pl.pallas_call(
        _mlp_body,
        grid=(grid,),
        in_specs=[
            pl.BlockSpec((_BLK, D_IN), lambda i: (i, 0)),
            pl.BlockSpec((D_IN, D_OUT), lambda i: (0, 0)),
            pl.BlockSpec((1, D_OUT), lambda i: (0, 0)),
            pl.BlockSpec((D_OUT, D_OUT), lambda i: (0, 0)),
            pl.BlockSpec((1, D_OUT), lambda i: (0, 0)),
        ],
        out_specs=pl.BlockSpec((_BLK, D_OUT), lambda i: (i, 0)),
        out_shape=jax.ShapeDtypeStruct((n, D_OUT), jnp.float32),
    )(freqs, W1, b1.reshape(1, D_OUT), W2, b2.reshape(1, D_OUT))


_NC, _NS = 2, 16
_NW = _NC * _NS
_TOK = 64 * 2048
_TPW = _TOK // _NW
_CH = 32
_NPAIR = _TPW // (2 * _CH)

@functools.cache
def _make_gather():
    mesh = plsc.VectorSubcoreMesh(core_axis_name="c", subcore_axis_name="s")

    @functools.partial(
        pl.kernel,
        out_type=jax.ShapeDtypeStruct((_TOK, D_OUT), jnp.float32),
        mesh=mesh,
        scratch_types=[
            pltpu.VMEM((_TPW,), jnp.int32),
            pltpu.VMEM((_CH, D_OUT), jnp.float32),
            pltpu.VMEM((_CH, D_OUT), jnp.float32),
            pltpu.SemaphoreType.DMA,
            pltpu.SemaphoreType.DMA,
        ],
    )
    def _gather(table_hbm, idx_hbm, out_hbm, idx_v, buf0, buf1, sem0, sem1):
        wid = lax.axis_index("s") * _NC + lax.axis_index("c")
        base = wid * _TPW
        pltpu.sync_copy(idx_hbm.at[pl.ds(base, _TPW)], idx_v)

        def body(jj, carry):
            o0 = jj * (2 * _CH)
            o1 = o0 + _CH
            c0 = pltpu.async_copy(table_hbm.at[idx_v.at[pl.ds(o0, _CH)]], buf0, sem0)
            c1 = pltpu.async_copy(table_hbm.at[idx_v.at[pl.ds(o1, _CH)]], buf1, sem1)
            c0.wait()
            pltpu.sync_copy(buf0, out_hbm.at[pl.ds(base + o0, _CH)])
            c1.wait()
            pltpu.sync_copy(buf1, out_hbm.at[pl.ds(base + o1, _CH)])
            return carry

        lax.fori_loop(0, _NPAIR, body, 0)

    return _gather


def kernel(orders, freqs_cis, W1, b1, W2, b2):
    table = _mlp_table(freqs_cis, W1, b1, W2, b2)
    flat = orders.reshape(-1)
    out = _make_gather()(table, flat)
    return out.reshape(orders.shape[0], orders.shape[1], D_OUT)
